# baseline (speedup 1.0000x reference)
import functools

import jax
import jax.numpy as jnp
from jax import lax
from jax.experimental import pallas as pl
from jax.experimental.pallas import tpu as pltpu
from jax.experimental.pallas import tpu_sc as plsc


def _lane_shuffle(v, idx16):
    dn = lax.GatherDimensionNumbers(offset_dims=(), collapsed_slice_dims=(0,),
                                    start_index_map=(0,))
    return lax.gather(v, idx16[:, None], dimension_numbers=dn,
                      slice_sizes=(1,),
                      mode=lax.GatherScatterMode.PROMISE_IN_BOUNDS)


_N = 10000
_D = 128
_E = 320000
_L = 3
_NC = 2
_NS = 16
_NW = _NC * _NS
_EPW = _E // _NW
_B = 80
_NCHUNK = _EPW // _B
_ZCH = 40
_NZCH = _N // _ZCH


@functools.partial(
    pl.kernel,
    out_type=(jax.ShapeDtypeStruct((_NC, _N, _D), jnp.float32),
              jax.ShapeDtypeStruct((_NC, _NS, _N), jnp.float32)),
    mesh=plsc.VectorSubcoreMesh(core_axis_name="c", subcore_axis_name="s"),
    scratch_types=[
        pltpu.VMEM((_B,), jnp.int32),
        pltpu.VMEM((_B,), jnp.int32),
        pltpu.VMEM((_B, _D), jnp.float32),
        pltpu.VMEM((_B, _D), jnp.float32),
        pltpu.VMEM((_D,), jnp.float32),
        pltpu.VMEM((_ZCH, _D), jnp.float32),
        pltpu.VMEM((_N,), jnp.float32),
        pltpu.VMEM_SHARED((_N, _D), jnp.float32),
        pltpu.SemaphoreType.DMA,
        pltpu.SemaphoreType.DMA,
    ],
)
def _edge_kernel(xl_hbm, xr_hbm, src_hbm, dst_hbm, att_hbm,
                 out_hbm, den_hbm,
                 src_v, dst_v, xlr_v, xrr_v, att_v, zbuf_v, den_v,
                 acc_sh, sem1, sem2):
    c = lax.axis_index("c")
    s = lax.axis_index("s")
    wid = s * _NC + c

    pltpu.sync_copy(att_hbm, att_v)

    z16 = jnp.zeros((16,), jnp.float32)

    def _zrow(i, carry):
        for k in range(_D // 16):
            zbuf_v[i, pl.ds(k * 16, 16)] = z16
        return carry
    lax.fori_loop(0, _ZCH, _zrow, 0)

    def _zden(i, carry):
        den_v[pl.ds(i * 16, 16)] = z16
        return carry
    lax.fori_loop(0, _N // 16, _zden, 0)

    for k in range((_NZCH + _NS - 1) // _NS):
        idx = s + _NS * k
        @pl.when(idx < _NZCH)
        def _():
            pltpu.sync_copy(zbuf_v, acc_sh.at[pl.ds(idx * _ZCH, _ZCH)])
    plsc.subcore_barrier()

    def _chunk(ci, carry):
        base = wid * _EPW + ci * _B
        pltpu.sync_copy(src_hbm.at[pl.ds(base, _B)], src_v)
        pltpu.sync_copy(dst_hbm.at[pl.ds(base, _B)], dst_v)
        cp1 = pltpu.async_copy(xl_hbm.at[src_v], xlr_v, sem1)
        cp2 = pltpu.async_copy(xr_hbm.at[dst_v], xrr_v, sem2)
        cp1.wait()
        cp2.wait()

        lanes = lax.iota(jnp.int32, 16)
        for gi in range(_B // 16):
            d16 = dst_v[pl.ds(gi * 16, 16)]
            for eo in range(16):
                e = gi * 16 + eo
                acc = jnp.zeros((16,), jnp.float32)
                a_parts = []
                for k in range(_D // 16):
                    a = xlr_v[e, pl.ds(k * 16, 16)]
                    b = xrr_v[e, pl.ds(k * 16, 16)]
                    a_parts.append(a)
                    v = a + b
                    lk = jnp.maximum(v, 0.2 * v)
                    acc = acc + lk * att_v[pl.ds(k * 16, 16)]
                for off in (8, 4, 2, 1):
                    acc = acc + _lane_shuffle(acc, lanes ^ off)
                wv = jnp.exp(acc)
                for k in range(_D // 16):
                    xlr_v[e, pl.ds(k * 16, 16)] = a_parts[k] * wv
                d = d16[eo]
                dbase = d & ~15
                lane = d & 15
                sel = jnp.where(lanes == lane, wv, 0.0)
                plsc.addupdate(den_v.at[pl.ds(dbase, 16)], sel)

        pltpu.sync_copy(xlr_v, acc_sh.at[dst_v], add=True)
        return carry
    lax.fori_loop(0, _NCHUNK, _chunk, 0)

    plsc.subcore_barrier()
    for k in range((_NZCH + _NS - 1) // _NS):
        idx = s + _NS * k
        @pl.when(idx < _NZCH)
        def _():
            r0 = idx * _ZCH
            pltpu.sync_copy(acc_sh.at[pl.ds(r0, _ZCH)],
                            out_hbm.at[c, pl.ds(r0, _ZCH)])
    pltpu.sync_copy(den_v, den_hbm.at[c, s])


def _layer_norm(h, g, b):
    mu = jnp.mean(h, axis=1, keepdims=True)
    d = h - mu
    var = jnp.mean(d * d, axis=1, keepdims=True)
    return d * lax.rsqrt(var + 1e-5) * g + b


def _erf(z):
    az = jnp.abs(z)
    t = 1.0 / (1.0 + 0.3275911 * az)
    poly = ((((1.061405429 * t - 1.453152027) * t + 1.421413741) * t
             - 0.284496736) * t + 0.254829592) * t
    return jnp.sign(z) * (1.0 - poly * jnp.exp(-az * az))


def _combine(p_ref, den_ref, h_ref, bias_ref):
    num = p_ref[0] + p_ref[1]
    den = jnp.sum(den_ref[...].reshape(_NW, _N), axis=0)
    t = num / (den[:, None] + 1e-16)
    t = t + bias_ref[...]
    z = t * 0.7071067811865476
    g = t * 0.5 * (1.0 + _erf(z))
    return g + h_ref[...]


def _pre_body(h_ref, g_ref, b_ref, wlT_ref, bl_ref, wrT_ref, br_ref,
              xl_ref, xr_ref):
    z = _layer_norm(h_ref[...], g_ref[...], b_ref[...])
    xl_ref[...] = jnp.dot(z, wlT_ref[...],
                          preferred_element_type=jnp.float32) + bl_ref[...]
    xr_ref[...] = jnp.dot(z, wrT_ref[...],
                          preferred_element_type=jnp.float32) + br_ref[...]


def _mid_body(p_ref, den_ref, h_ref, bias_ref, g_ref, b_ref, wlT_ref, bl_ref,
              wrT_ref, br_ref, hn_ref, xl_ref, xr_ref):
    hn = _combine(p_ref, den_ref, h_ref, bias_ref)
    hn_ref[...] = hn
    z = _layer_norm(hn, g_ref[...], b_ref[...])
    xl_ref[...] = jnp.dot(z, wlT_ref[...],
                          preferred_element_type=jnp.float32) + bl_ref[...]
    xr_ref[...] = jnp.dot(z, wrT_ref[...],
                          preferred_element_type=jnp.float32) + br_ref[...]


def _post_body(p_ref, den_ref, h_ref, bias_ref, wpT_ref, bp_ref, out_ref):
    hn = _combine(p_ref, den_ref, h_ref, bias_ref)
    out_ref[...] = jnp.dot(hn, wpT_ref[...],
                           preferred_element_type=jnp.float32) + bp_ref[...]


_f32 = jnp.float32
_pre_call = pl.pallas_call(
    _pre_body,
    out_shape=(jax.ShapeDtypeStruct((_N, _D), _f32),
               jax.ShapeDtypeStruct((_N, _D), _f32)),
)
_mid_call = pl.pallas_call(
    _mid_body,
    out_shape=(jax.ShapeDtypeStruct((_N, _D), _f32),
               jax.ShapeDtypeStruct((_N, _D), _f32),
               jax.ShapeDtypeStruct((_N, _D), _f32)),
)
_post_call = pl.pallas_call(
    _post_body,
    out_shape=jax.ShapeDtypeStruct((_N, _D), _f32),
)


def kernel(x, edge_index, emb, norm_gamma, norm_beta, Wl, bl, Wr, br, att,
           gat_bias, W_post, b_post):
    src = edge_index[0].astype(jnp.int32)
    dst = edge_index[1].astype(jnp.int32)
    WlT = jnp.swapaxes(Wl, 1, 2)
    WrT = jnp.swapaxes(Wr, 1, 2)
    g = norm_gamma.reshape(_L, 1, _D)
    b = norm_beta.reshape(_L, 1, _D)
    bl2 = bl.reshape(_L, 1, _D)
    br2 = br.reshape(_L, 1, _D)
    gb = gat_bias.reshape(_L, 1, _D)
    bp = b_post.reshape(1, _D)

    xl, xr = _pre_call(emb, g[0], b[0], WlT[0], bl2[0], WrT[0], br2[0])
    p, den = _edge_kernel(xl, xr, src, dst, att[0])
    h1, xl, xr = _mid_call(p, den, emb, gb[0], g[1], b[1], WlT[1], bl2[1],
                           WrT[1], br2[1])
    p, den = _edge_kernel(xl, xr, src, dst, att[1])
    h2, xl, xr = _mid_call(p, den, h1, gb[1], g[2], b[2], WlT[2], bl2[2],
                           WrT[2], br2[2])
    p, den = _edge_kernel(xl, xr, src, dst, att[2])
    return _post_call(p, den, h2, gb[2], W_post.T, bp)


# speedup vs baseline: 1.7728x; 1.7728x over previous
import functools

import jax
import jax.numpy as jnp
from jax import lax
from jax.experimental import pallas as pl
from jax.experimental.pallas import tpu as pltpu
from jax.experimental.pallas import tpu_sc as plsc

_N = 10000
_D = 128
_E = 320000
_L = 3
_NC = 2
_NS = 16
_NW = _NC * _NS
_EPW = _E // _NW
_B = 80
_NCHUNK = _EPW // _B
_ZCH = 40
_NZCH = _N // _ZCH
_DR = 1280


def _lane_shuffle(v, idx16):
    dn = lax.GatherDimensionNumbers(offset_dims=(), collapsed_slice_dims=(0,),
                                    start_index_map=(0,))
    return lax.gather(v, idx16[:, None], dimension_numbers=dn,
                      slice_sizes=(1,),
                      mode=lax.GatherScatterMode.PROMISE_IN_BOUNDS)


@functools.partial(
    pl.kernel,
    out_type=(jax.ShapeDtypeStruct((_NC, _N, _D), jnp.float32),
              jax.ShapeDtypeStruct((_NC, _DR, _D), jnp.float32)),
    mesh=plsc.VectorSubcoreMesh(core_axis_name="c", subcore_axis_name="s"),
    scratch_types=[
        pltpu.VMEM((_B,), jnp.int32),
        pltpu.VMEM((_B,), jnp.int32),
        pltpu.VMEM((_B,), jnp.int32),
        pltpu.VMEM((_B,), jnp.int32),
        pltpu.VMEM((_B,), jnp.int32),
        pltpu.VMEM((_B,), jnp.int32),
        pltpu.VMEM((_B, _D), jnp.float32),
        pltpu.VMEM((_B, _D), jnp.float32),
        pltpu.VMEM((_B, _D), jnp.float32),
        pltpu.VMEM((_D,), jnp.float32),
        pltpu.VMEM((_ZCH, _D), jnp.float32),
        pltpu.VMEM_SHARED((_N, _D), jnp.float32),
        pltpu.VMEM_SHARED((_DR, _D), jnp.float32),
        pltpu.SemaphoreType.DMA,
        pltpu.SemaphoreType.DMA,
        pltpu.SemaphoreType.DMA,
        pltpu.SemaphoreType.DMA,
    ],
)
def _edge_kernel(xl_hbm, xr_hbm, src_hbm, dst_hbm, att_hbm,
                 out_hbm, den_hbm,
                 srcA_v, dstA_v, dsthA_v, srcB_v, dstB_v, dsthB_v,
                 xlr_v, xrr_v, dbuf_v, att_v, zbuf_v,
                 acc_sh, den_sh, sem1, sem2, sem3, sem4):
    c = lax.axis_index("c")
    s = lax.axis_index("s")
    wid = s * _NC + c

    pltpu.sync_copy(att_hbm, att_v)

    z16 = jnp.zeros((16,), jnp.float32)

    def _zrow(i, carry):
        for k in range(_D // 16):
            zbuf_v[i, pl.ds(k * 16, 16)] = z16
        return carry
    lax.fori_loop(0, _ZCH, _zrow, 0)

    def _zdrow(i, carry):
        for k in range(_D // 16):
            dbuf_v[i, pl.ds(k * 16, 16)] = z16
        return carry
    lax.fori_loop(0, _B, _zdrow, 0)

    for k in range((_NZCH + _NS - 1) // _NS):
        idx = s + _NS * k
        @pl.when(idx < _NZCH)
        def _():
            pltpu.sync_copy(zbuf_v, acc_sh.at[pl.ds(idx * _ZCH, _ZCH)])
    for k in range((_DR // _NS) // _ZCH):
        pltpu.sync_copy(
            zbuf_v,
            den_sh.at[pl.ds(s * (_DR // _NS) + k * _ZCH, _ZCH)])
    plsc.subcore_barrier()

    lanes = lax.iota(jnp.int32, 16)
    ebase = wid * _EPW

    def _load_idx(ci, src_v, dst_v):
        base = ebase + ci * _B
        cpa = pltpu.async_copy(src_hbm.at[pl.ds(base, _B)], src_v, sem1)
        cpb = pltpu.async_copy(dst_hbm.at[pl.ds(base, _B)], dst_v, sem2)
        cpa.wait()
        cpb.wait()

    def _gather(src_v, dst_v):
        cp1 = pltpu.async_copy(xl_hbm.at[src_v], xlr_v, sem1)
        cp2 = pltpu.async_copy(xr_hbm.at[dst_v], xrr_v, sem2)
        cp1.wait()
        cp2.wait()

    def _compute(dst_v, dsth_v):
        def _group(gi, gcarry):
            d16 = dst_v[pl.ds(gi * 16, 16)]
            dsth_v[pl.ds(gi * 16, 16)] = d16 >> 3
            for eo in range(16):
                e = gi * 16 + eo
                acc = jnp.zeros((16,), jnp.float32)
                a_parts = []
                for k in range(_D // 16):
                    a = xlr_v[e, pl.ds(k * 16, 16)]
                    b = xrr_v[e, pl.ds(k * 16, 16)]
                    a_parts.append(a)
                    v = a + b
                    lk = jnp.maximum(v, 0.2 * v)
                    acc = acc + lk * att_v[pl.ds(k * 16, 16)]
                for off in (8, 4, 2, 1):
                    acc = acc + _lane_shuffle(acc, lanes ^ off)
                wv = jnp.exp(acc)
                for k in range(_D // 16):
                    xlr_v[e, pl.ds(k * 16, 16)] = a_parts[k] * wv
                col = (d16[eo] & 7) * 16
                dbuf_v[e, pl.ds(col, 16)] = wv
            return gcarry
        lax.fori_loop(0, _B // 16, _group, 0)

    def _scatter_start(dst_v, dsth_v):
        cpm = pltpu.async_copy(xlr_v, acc_sh.at[dst_v], sem3, add=True)
        cpd = pltpu.async_copy(dbuf_v, den_sh.at[dsth_v], sem4, add=True)
        return cpm, cpd

    def _rezero(dst_v):
        def _gz(gi, gcarry):
            d16 = dst_v[pl.ds(gi * 16, 16)]
            for eo in range(16):
                col = (d16[eo] & 7) * 16
                dbuf_v[gi * 16 + eo, pl.ds(col, 16)] = z16
            return gcarry
        lax.fori_loop(0, _B // 16, _gz, 0)

    _load_idx(0, srcA_v, dstA_v)

    def _pair(pi, carry):
        c0 = 2 * pi
        _gather(srcA_v, dstA_v)
        _compute(dstA_v, dsthA_v)
        cpm, cpd = _scatter_start(dstA_v, dsthA_v)
        _load_idx(c0 + 1, srcB_v, dstB_v)
        cpm.wait()
        cpd.wait()
        _rezero(dstA_v)
        _gather(srcB_v, dstB_v)
        _compute(dstB_v, dsthB_v)
        cpm, cpd = _scatter_start(dstB_v, dsthB_v)
        _load_idx(c0 + 2, srcA_v, dstA_v)
        cpm.wait()
        cpd.wait()
        _rezero(dstB_v)
        return carry
    lax.fori_loop(0, (_NCHUNK - 1) // 2, _pair, 0)

    _gather(srcA_v, dstA_v)
    _compute(dstA_v, dsthA_v)
    cpm, cpd = _scatter_start(dstA_v, dsthA_v)
    cpm.wait()
    cpd.wait()

    plsc.subcore_barrier()
    for k in range((_NZCH + _NS - 1) // _NS):
        idx = s + _NS * k
        @pl.when(idx < _NZCH)
        def _():
            r0 = idx * _ZCH
            pltpu.sync_copy(acc_sh.at[pl.ds(r0, _ZCH)],
                            out_hbm.at[c, pl.ds(r0, _ZCH)])
    dpt = _DR // _NS
    pltpu.sync_copy(den_sh.at[pl.ds(s * dpt, dpt)],
                    den_hbm.at[c, pl.ds(s * dpt, dpt)])


def _layer_norm(h, g, b):
    mu = jnp.mean(h, axis=1, keepdims=True)
    d = h - mu
    var = jnp.mean(d * d, axis=1, keepdims=True)
    return d * lax.rsqrt(var + 1e-5) * g + b


def _erf(z):
    az = jnp.abs(z)
    t = 1.0 / (1.0 + 0.3275911 * az)
    poly = ((((1.061405429 * t - 1.453152027) * t + 1.421413741) * t
             - 0.284496736) * t + 0.254829592) * t
    return jnp.sign(z) * (1.0 - poly * jnp.exp(-az * az))


def _combine(p_ref, den_ref, h_ref, bias_ref):
    num = (p_ref[0] + p_ref[1]).reshape(_N // 8, 8, _D)
    den_rows = den_ref[0] + den_ref[1]
    cc = lax.broadcasted_iota(jnp.int32, (_D, 8), 0)
    kk = lax.broadcasted_iota(jnp.int32, (_D, 8), 1)
    sel = (cc == kk * 16).astype(jnp.float32)
    den = jnp.dot(den_rows, sel,
                  preferred_element_type=jnp.float32)[:_N // 8, :]
    t = num / (den[:, :, None] + 1e-16)
    t = t.reshape(_N, _D) + bias_ref[...]
    z = t * 0.7071067811865476
    g = t * 0.5 * (1.0 + _erf(z))
    return g + h_ref[...]


def _pre_body(h_ref, g_ref, b_ref, wlT_ref, bl_ref, wrT_ref, br_ref,
              xl_ref, xr_ref):
    z = _layer_norm(h_ref[...], g_ref[...], b_ref[...])
    xl_ref[...] = jnp.dot(z, wlT_ref[...],
                          preferred_element_type=jnp.float32) + bl_ref[...]
    xr_ref[...] = jnp.dot(z, wrT_ref[...],
                          preferred_element_type=jnp.float32) + br_ref[...]


def _mid_body(p_ref, den_ref, h_ref, bias_ref, g_ref, b_ref, wlT_ref, bl_ref,
              wrT_ref, br_ref, hn_ref, xl_ref, xr_ref):
    hn = _combine(p_ref, den_ref, h_ref, bias_ref)
    hn_ref[...] = hn
    z = _layer_norm(hn, g_ref[...], b_ref[...])
    xl_ref[...] = jnp.dot(z, wlT_ref[...],
                          preferred_element_type=jnp.float32) + bl_ref[...]
    xr_ref[...] = jnp.dot(z, wrT_ref[...],
                          preferred_element_type=jnp.float32) + br_ref[...]


def _post_body(p_ref, den_ref, h_ref, bias_ref, wpT_ref, bp_ref, out_ref):
    hn = _combine(p_ref, den_ref, h_ref, bias_ref)
    out_ref[...] = jnp.dot(hn, wpT_ref[...],
                           preferred_element_type=jnp.float32) + bp_ref[...]


_f32 = jnp.float32
_pre_call = pl.pallas_call(
    _pre_body,
    out_shape=(jax.ShapeDtypeStruct((_N, _D), _f32),
               jax.ShapeDtypeStruct((_N, _D), _f32)),
)
_mid_call = pl.pallas_call(
    _mid_body,
    out_shape=(jax.ShapeDtypeStruct((_N, _D), _f32),
               jax.ShapeDtypeStruct((_N, _D), _f32),
               jax.ShapeDtypeStruct((_N, _D), _f32)),
)
_post_call = pl.pallas_call(
    _post_body,
    out_shape=jax.ShapeDtypeStruct((_N, _D), _f32),
)


def kernel(x, edge_index, emb, norm_gamma, norm_beta, Wl, bl, Wr, br, att,
           gat_bias, W_post, b_post):
    src = edge_index[0].astype(jnp.int32)
    dst = edge_index[1].astype(jnp.int32)
    WlT = jnp.swapaxes(Wl, 1, 2)
    WrT = jnp.swapaxes(Wr, 1, 2)
    g = norm_gamma.reshape(_L, 1, _D)
    b = norm_beta.reshape(_L, 1, _D)
    bl2 = bl.reshape(_L, 1, _D)
    br2 = br.reshape(_L, 1, _D)
    gb = gat_bias.reshape(_L, 1, _D)
    bp = b_post.reshape(1, _D)

    xl, xr = _pre_call(emb, g[0], b[0], WlT[0], bl2[0], WrT[0], br2[0])
    p, den = _edge_kernel(xl, xr, src, dst, att[0])
    h1, xl, xr = _mid_call(p, den, emb, gb[0], g[1], b[1], WlT[1], bl2[1],
                           WrT[1], br2[1])
    p, den = _edge_kernel(xl, xr, src, dst, att[1])
    h2, xl, xr = _mid_call(p, den, h1, gb[1], g[2], b[2], WlT[2], bl2[2],
                           WrT[2], br2[2])
    p, den = _edge_kernel(xl, xr, src, dst, att[2])
    return _post_call(p, den, h2, gb[2], W_post.T, bp)


# speedup vs baseline: 1.8497x; 1.0433x over previous
import functools

import jax
import jax.numpy as jnp
from jax import lax
from jax.experimental import pallas as pl
from jax.experimental.pallas import tpu as pltpu
from jax.experimental.pallas import tpu_sc as plsc

_N = 10000
_D = 128
_E = 320000
_L = 3
_NC = 2
_NS = 16
_NW = _NC * _NS
_EPW = _E // _NW
_B = 48
_NCHUNK = _EPW // _B
_NPAIR = _NCHUNK // 2
_TAIL = _EPW - _NCHUNK * _B
_ZCH = 16
_NZCH = _N // _ZCH
_DR = 1280


def _lane_shuffle(v, idx16):
    dn = lax.GatherDimensionNumbers(offset_dims=(), collapsed_slice_dims=(0,),
                                    start_index_map=(0,))
    return lax.gather(v, idx16[:, None], dimension_numbers=dn,
                      slice_sizes=(1,),
                      mode=lax.GatherScatterMode.PROMISE_IN_BOUNDS)


@functools.partial(
    pl.kernel,
    out_type=(jax.ShapeDtypeStruct((_NC, _N, _D), jnp.float32),
              jax.ShapeDtypeStruct((_NC, _DR, _D), jnp.float32)),
    mesh=plsc.VectorSubcoreMesh(core_axis_name="c", subcore_axis_name="s"),
    scratch_types=[
        pltpu.VMEM((_B,), jnp.int32),
        pltpu.VMEM((_B,), jnp.int32),
        pltpu.VMEM((_B,), jnp.int32),
        pltpu.VMEM((_B,), jnp.int32),
        pltpu.VMEM((_B,), jnp.int32),
        pltpu.VMEM((_B,), jnp.int32),
        pltpu.VMEM((_B, _D), jnp.float32),
        pltpu.VMEM((_B, _D), jnp.float32),
        pltpu.VMEM((_B, _D), jnp.float32),
        pltpu.VMEM((_B, _D), jnp.float32),
        pltpu.VMEM((_B, _D), jnp.float32),
        pltpu.VMEM((_B, _D), jnp.float32),
        pltpu.VMEM((_D,), jnp.float32),
        pltpu.VMEM((_ZCH, _D), jnp.float32),
        pltpu.VMEM_SHARED((_N, _D), jnp.float32),
        pltpu.VMEM_SHARED((_DR, _D), jnp.float32),
        pltpu.SemaphoreType.DMA,
        pltpu.SemaphoreType.DMA,
        pltpu.SemaphoreType.DMA,
        pltpu.SemaphoreType.DMA,
        pltpu.SemaphoreType.DMA,
        pltpu.SemaphoreType.DMA,
        pltpu.SemaphoreType.DMA,
        pltpu.SemaphoreType.DMA,
        pltpu.SemaphoreType.DMA,
        pltpu.SemaphoreType.DMA,
    ],
)
def _edge_kernel(xl_hbm, xr_hbm, src_hbm, dst_hbm, att_hbm,
                 out_hbm, den_hbm,
                 srcA_v, dstA_v, dsthA_v, srcB_v, dstB_v, dsthB_v,
                 xlrA_v, xrrA_v, dbufA_v, xlrB_v, xrrB_v, dbufB_v,
                 att_v, zbuf_v, acc_sh, den_sh,
                 semi1, semi2, semgA1, semgA2, semgB1, semgB2,
                 semsA1, semsA2, semsB1, semsB2):
    c = lax.axis_index("c")
    s = lax.axis_index("s")
    wid = s * _NC + c

    pltpu.sync_copy(att_hbm, att_v)

    z16 = jnp.zeros((16,), jnp.float32)

    def _zrow(i, carry):
        for k in range(_D // 16):
            zbuf_v[i, pl.ds(k * 16, 16)] = z16
        return carry
    lax.fori_loop(0, _ZCH, _zrow, 0)

    def _zdrow(i, carry):
        for k in range(_D // 16):
            dbufA_v[i, pl.ds(k * 16, 16)] = z16
            dbufB_v[i, pl.ds(k * 16, 16)] = z16
        return carry
    lax.fori_loop(0, _B, _zdrow, 0)

    for k in range((_NZCH + _NS - 1) // _NS):
        idx = s + _NS * k
        @pl.when(idx < _NZCH)
        def _():
            pltpu.sync_copy(zbuf_v, acc_sh.at[pl.ds(idx * _ZCH, _ZCH)])
    for k in range((_DR // _NS) // _ZCH):
        pltpu.sync_copy(
            zbuf_v,
            den_sh.at[pl.ds(s * (_DR // _NS) + k * _ZCH, _ZCH)])
    plsc.subcore_barrier()

    lanes = lax.iota(jnp.int32, 16)
    ebase = wid * _EPW

    def _load_idx(base, src_v, dst_v):
        cpa = pltpu.async_copy(src_hbm.at[pl.ds(base, _B)], src_v, semi1)
        cpb = pltpu.async_copy(dst_hbm.at[pl.ds(base, _B)], dst_v, semi2)
        cpa.wait()
        cpb.wait()

    def _gather_start(src_v, dst_v, xlr_v, xrr_v, sg1, sg2):
        pltpu.async_copy(xl_hbm.at[src_v], xlr_v, sg1)
        pltpu.async_copy(xr_hbm.at[dst_v], xrr_v, sg2)

    def _gather_wait(src_v, dst_v, xlr_v, xrr_v, sg1, sg2):
        pltpu.make_async_copy(xl_hbm.at[src_v], xlr_v, sg1).wait()
        pltpu.make_async_copy(xr_hbm.at[dst_v], xrr_v, sg2).wait()

    def _compute(dst_v, dsth_v, xlr_v, xrr_v, dbuf_v):
        def _group(gi, gcarry):
            d16 = dst_v[pl.ds(gi * 16, 16)]
            dsth_v[pl.ds(gi * 16, 16)] = d16 >> 3
            for eo in range(16):
                e = gi * 16 + eo
                acc = jnp.zeros((16,), jnp.float32)
                a_parts = []
                for k in range(_D // 16):
                    a = xlr_v[e, pl.ds(k * 16, 16)]
                    b = xrr_v[e, pl.ds(k * 16, 16)]
                    a_parts.append(a)
                    v = a + b
                    lk = jnp.maximum(v, 0.2 * v)
                    acc = acc + lk * att_v[pl.ds(k * 16, 16)]
                for off in (8, 4, 2, 1):
                    acc = acc + _lane_shuffle(acc, lanes ^ off)
                wv = jnp.exp(acc)
                for k in range(_D // 16):
                    xlr_v[e, pl.ds(k * 16, 16)] = a_parts[k] * wv
                col = (d16[eo] & 7) * 16
                dbuf_v[e, pl.ds(col, 16)] = wv
            return gcarry
        lax.fori_loop(0, _B // 16, _group, 0)

    def _scatter_start(dst_v, dsth_v, xlr_v, dbuf_v, ss1, ss2):
        pltpu.async_copy(xlr_v, acc_sh.at[dst_v], ss1, add=True)
        pltpu.async_copy(dbuf_v, den_sh.at[dsth_v], ss2, add=True)

    def _scatter_wait(dst_v, dsth_v, xlr_v, dbuf_v, ss1, ss2):
        pltpu.make_async_copy(xlr_v, acc_sh.at[dst_v], ss1).wait()
        pltpu.make_async_copy(dbuf_v, den_sh.at[dsth_v], ss2).wait()

    def _rezero(dst_v, dbuf_v):
        def _gz(gi, gcarry):
            d16 = dst_v[pl.ds(gi * 16, 16)]
            for eo in range(16):
                col = (d16[eo] & 7) * 16
                dbuf_v[gi * 16 + eo, pl.ds(col, 16)] = z16
            return gcarry
        lax.fori_loop(0, _B // 16, _gz, 0)

    _load_idx(ebase, srcA_v, dstA_v)
    _gather_start(srcA_v, dstA_v, xlrA_v, xrrA_v, semgA1, semgA2)

    def _pair(pi, carry):
        c0 = 2 * pi
        @pl.when(pi > 0)
        def _():
            _scatter_wait(dstB_v, dsthB_v, xlrB_v, dbufB_v, semsB1, semsB2)
            _rezero(dstB_v, dbufB_v)
        _load_idx(ebase + (c0 + 1) * _B, srcB_v, dstB_v)
        _gather_start(srcB_v, dstB_v, xlrB_v, xrrB_v, semgB1, semgB2)
        _gather_wait(srcA_v, dstA_v, xlrA_v, xrrA_v, semgA1, semgA2)
        _compute(dstA_v, dsthA_v, xlrA_v, xrrA_v, dbufA_v)
        _scatter_start(dstA_v, dsthA_v, xlrA_v, dbufA_v, semsA1, semsA2)
        _gather_wait(srcB_v, dstB_v, xlrB_v, xrrB_v, semgB1, semgB2)
        _compute(dstB_v, dsthB_v, xlrB_v, xrrB_v, dbufB_v)
        _scatter_start(dstB_v, dsthB_v, xlrB_v, dbufB_v, semsB1, semsB2)
        _scatter_wait(dstA_v, dsthA_v, xlrA_v, dbufA_v, semsA1, semsA2)
        _rezero(dstA_v, dbufA_v)
        @pl.when(pi < _NPAIR - 1)
        def _():
            _load_idx(ebase + (c0 + 2) * _B, srcA_v, dstA_v)
            _gather_start(srcA_v, dstA_v, xlrA_v, xrrA_v, semgA1, semgA2)
        return carry
    lax.fori_loop(0, _NPAIR, _pair, 0)

    _scatter_wait(dstB_v, dsthB_v, xlrB_v, dbufB_v, semsB1, semsB2)
    _rezero(dstB_v, dbufB_v)

    _load_idx(ebase + _EPW - _B, srcA_v, dstA_v)
    _gather_start(srcA_v, dstA_v, xlrA_v, xrrA_v, semgA1, semgA2)
    _gather_wait(srcA_v, dstA_v, xlrA_v, xrrA_v, semgA1, semgA2)
    _compute(dstA_v, dsthA_v, xlrA_v, xrrA_v, dbufA_v)

    def _ztail(i, carry):
        for k in range(_D // 16):
            xlrA_v[i, pl.ds(k * 16, 16)] = z16
            dbufA_v[i, pl.ds(k * 16, 16)] = z16
        return carry
    lax.fori_loop(0, _B - _TAIL, _ztail, 0)
    _scatter_start(dstA_v, dsthA_v, xlrA_v, dbufA_v, semsA1, semsA2)
    _scatter_wait(dstA_v, dsthA_v, xlrA_v, dbufA_v, semsA1, semsA2)

    plsc.subcore_barrier()
    for k in range((_NZCH + _NS - 1) // _NS):
        idx = s + _NS * k
        @pl.when(idx < _NZCH)
        def _():
            r0 = idx * _ZCH
            pltpu.sync_copy(acc_sh.at[pl.ds(r0, _ZCH)],
                            out_hbm.at[c, pl.ds(r0, _ZCH)])
    dpt = _DR // _NS
    pltpu.sync_copy(den_sh.at[pl.ds(s * dpt, dpt)],
                    den_hbm.at[c, pl.ds(s * dpt, dpt)])


def _layer_norm(h, g, b):
    mu = jnp.mean(h, axis=1, keepdims=True)
    d = h - mu
    var = jnp.mean(d * d, axis=1, keepdims=True)
    return d * lax.rsqrt(var + 1e-5) * g + b


def _erf(z):
    az = jnp.abs(z)
    t = 1.0 / (1.0 + 0.3275911 * az)
    poly = ((((1.061405429 * t - 1.453152027) * t + 1.421413741) * t
             - 0.284496736) * t + 0.254829592) * t
    return jnp.sign(z) * (1.0 - poly * jnp.exp(-az * az))


def _combine(p_ref, den_ref, h_ref, bias_ref):
    num = (p_ref[0] + p_ref[1]).reshape(_N // 8, 8, _D)
    den_rows = den_ref[0] + den_ref[1]
    cc = lax.broadcasted_iota(jnp.int32, (_D, 8), 0)
    kk = lax.broadcasted_iota(jnp.int32, (_D, 8), 1)
    sel = (cc == kk * 16).astype(jnp.float32)
    den = jnp.dot(den_rows, sel,
                  preferred_element_type=jnp.float32)[:_N // 8, :]
    t = num / (den[:, :, None] + 1e-16)
    t = t.reshape(_N, _D) + bias_ref[...]
    z = t * 0.7071067811865476
    g = t * 0.5 * (1.0 + _erf(z))
    return g + h_ref[...]


def _pre_body(h_ref, g_ref, b_ref, wlT_ref, bl_ref, wrT_ref, br_ref,
              xl_ref, xr_ref):
    z = _layer_norm(h_ref[...], g_ref[...], b_ref[...])
    xl_ref[...] = jnp.dot(z, wlT_ref[...],
                          preferred_element_type=jnp.float32) + bl_ref[...]
    xr_ref[...] = jnp.dot(z, wrT_ref[...],
                          preferred_element_type=jnp.float32) + br_ref[...]


def _mid_body(p_ref, den_ref, h_ref, bias_ref, g_ref, b_ref, wlT_ref, bl_ref,
              wrT_ref, br_ref, hn_ref, xl_ref, xr_ref):
    hn = _combine(p_ref, den_ref, h_ref, bias_ref)
    hn_ref[...] = hn
    z = _layer_norm(hn, g_ref[...], b_ref[...])
    xl_ref[...] = jnp.dot(z, wlT_ref[...],
                          preferred_element_type=jnp.float32) + bl_ref[...]
    xr_ref[...] = jnp.dot(z, wrT_ref[...],
                          preferred_element_type=jnp.float32) + br_ref[...]


def _post_body(p_ref, den_ref, h_ref, bias_ref, wpT_ref, bp_ref, out_ref):
    hn = _combine(p_ref, den_ref, h_ref, bias_ref)
    out_ref[...] = jnp.dot(hn, wpT_ref[...],
                           preferred_element_type=jnp.float32) + bp_ref[...]


_f32 = jnp.float32
_pre_call = pl.pallas_call(
    _pre_body,
    out_shape=(jax.ShapeDtypeStruct((_N, _D), _f32),
               jax.ShapeDtypeStruct((_N, _D), _f32)),
)
_mid_call = pl.pallas_call(
    _mid_body,
    out_shape=(jax.ShapeDtypeStruct((_N, _D), _f32),
               jax.ShapeDtypeStruct((_N, _D), _f32),
               jax.ShapeDtypeStruct((_N, _D), _f32)),
)
_post_call = pl.pallas_call(
    _post_body,
    out_shape=jax.ShapeDtypeStruct((_N, _D), _f32),
)


def kernel(x, edge_index, emb, norm_gamma, norm_beta, Wl, bl, Wr, br, att,
           gat_bias, W_post, b_post):
    src = edge_index[0].astype(jnp.int32)
    dst = edge_index[1].astype(jnp.int32)
    WlT = jnp.swapaxes(Wl, 1, 2)
    WrT = jnp.swapaxes(Wr, 1, 2)
    g = norm_gamma.reshape(_L, 1, _D)
    b = norm_beta.reshape(_L, 1, _D)
    bl2 = bl.reshape(_L, 1, _D)
    br2 = br.reshape(_L, 1, _D)
    gb = gat_bias.reshape(_L, 1, _D)
    bp = b_post.reshape(1, _D)

    xl, xr = _pre_call(emb, g[0], b[0], WlT[0], bl2[0], WrT[0], br2[0])
    p, den = _edge_kernel(xl, xr, src, dst, att[0])
    h1, xl, xr = _mid_call(p, den, emb, gb[0], g[1], b[1], WlT[1], bl2[1],
                           WrT[1], br2[1])
    p, den = _edge_kernel(xl, xr, src, dst, att[1])
    h2, xl, xr = _mid_call(p, den, h1, gb[1], g[2], b[2], WlT[2], bl2[2],
                           WrT[2], br2[2])
    p, den = _edge_kernel(xl, xr, src, dst, att[2])
    return _post_call(p, den, h2, gb[2], W_post.T, bp)
